# baseline (speedup 1.0000x reference)
import functools

import jax
import jax.numpy as jnp
from jax import lax
from jax.experimental import pallas as pl
from jax.experimental.pallas import tpu as pltpu
from jax.experimental.pallas import tpu_sc as plsc

N_NODES = 10000
CH = 128
N_EDGES = 320000

NC = 2
NS = 16
NW = NC * NS
CHUNK = 128
NCH0 = 112
NCH1 = 48
CORE0_EDGES = NS * NCH0 * CHUNK
EP = NS * (NCH0 + NCH1) * CHUNK
NPAD = 10240
STRIPE = NPAD // NS
ROWB = 1024
GRID = NPAD // ROWB

DCH = 80
DTPW = DCH * CHUNK


def _sc_mesh():
    return plsc.VectorSubcoreMesh(core_axis_name="c", subcore_axis_name="s")


def _deg_body(dst_hbm, out_hbm, idxd_v, ones_v, zer_v, acc, sem):
    cid = lax.axis_index("c")
    sid = lax.axis_index("s")
    wid = cid * NS + sid

    for i in range(CHUNK // 16):
        ones_v[pl.ds(i * 16, 16)] = jnp.ones((16,), jnp.float32)
    for i in range(STRIPE // 16):
        zer_v[pl.ds(i * 16, 16)] = jnp.zeros((16,), jnp.float32)

    pltpu.sync_copy(zer_v, acc.at[pl.ds(sid * STRIPE, STRIPE)])
    plsc.subcore_barrier()

    base = wid * DTPW

    def chunk_step(j, carry):
        pltpu.sync_copy(dst_hbm.at[pl.ds(base + j * CHUNK, CHUNK)], idxd_v)
        pltpu.sync_copy(ones_v, acc.at[idxd_v], add=True)
        return carry

    lax.fori_loop(0, DCH, chunk_step, 0)
    plsc.subcore_barrier()
    pltpu.sync_copy(acc.at[pl.ds(sid * STRIPE, STRIPE)],
                    out_hbm.at[cid, pl.ds(sid * STRIPE, STRIPE)])


_deg_kernel = functools.partial(
    pl.kernel,
    out_type=jax.ShapeDtypeStruct((NC, NPAD), jnp.float32),
    mesh=_sc_mesh(),
    scratch_types=[
        pltpu.VMEM((CHUNK,), jnp.int32),
        pltpu.VMEM((CHUNK,), jnp.float32),
        pltpu.VMEM((STRIPE,), jnp.float32),
        pltpu.VMEM_SHARED((NPAD,), jnp.float32),
        pltpu.SemaphoreType.DMA,
    ],
)(_deg_body)


def _scat_body(g_hbm, src_hbm, dsti_hbm, z_hbm, out_hbm, is0, is1, idxd_v,
               r0, r1, acc, sg0, sg1, ss0, ss1):
    cid = lax.axis_index("c")
    sid = lax.axis_index("s")
    wid = cid * NS + sid

    pltpu.sync_copy(dsti_hbm.at[wid], idxd_v)

    gc = g_hbm.at[cid]

    @pl.when(cid == 0)
    def _():
        pltpu.sync_copy(gc.at[pl.ds(sid * STRIPE, STRIPE)],
                        acc.at[pl.ds(sid * STRIPE, STRIPE)])

    @pl.when(cid == 1)
    def _():
        pltpu.sync_copy(z_hbm, r0)
        for k in range(STRIPE // CHUNK):
            pltpu.sync_copy(
                r0, acc.at[pl.ds(sid * STRIPE + k * CHUNK, CHUNK)])

    plsc.subcore_barrier()

    isv = (is0, is1)
    rows = (r0, r1)
    sg = (sg0, sg1)
    ss = (ss0, ss1)
    nch = lax.select(cid == 0, NCH0, NCH1)
    base = cid * CORE0_EDGES + sid * nch * CHUNK

    def src_at(j):
        return src_hbm.at[pl.ds(base + j * CHUNK, CHUNK)]

    for h in range(2):
        pltpu.sync_copy(src_at(h), isv[h])
        pltpu.async_copy(gc.at[isv[h]], rows[h], sg[h])

    def pair_step(p, carry):
        for h in range(2):
            j = p * 2 + h

            @pl.when(j < nch)
            def _():
                pltpu.make_async_copy(gc.at[isv[h]], rows[h], sg[h]).wait()
                pltpu.async_copy(rows[h], acc.at[idxd_v.at[j]], ss[h],
                                 add=True)
                pltpu.sync_copy(src_at(j + 2), isv[h])
                pltpu.make_async_copy(rows[h], acc.at[idxd_v.at[j]],
                                      ss[h]).wait()
                pltpu.async_copy(gc.at[isv[h]], rows[h], sg[h])
        return carry

    lax.fori_loop(0, NCH0 // 2, pair_step, 0)
    for h in range(2):
        pltpu.make_async_copy(gc.at[isv[h]], rows[h], sg[h]).wait()

    plsc.subcore_barrier()
    pltpu.sync_copy(acc.at[pl.ds(sid * STRIPE, STRIPE)],
                    out_hbm.at[cid, pl.ds(sid * STRIPE, STRIPE)])


_scat_kernel = functools.partial(
    pl.kernel,
    out_type=jax.ShapeDtypeStruct((NC, NPAD, CH), jnp.float32),
    mesh=_sc_mesh(),
    scratch_types=[
        pltpu.VMEM((CHUNK,), jnp.int32),
        pltpu.VMEM((CHUNK,), jnp.int32),
        pltpu.VMEM((NCH0, CHUNK), jnp.int32),
        pltpu.VMEM((CHUNK, CH), jnp.float32),
        pltpu.VMEM((CHUNK, CH), jnp.float32),
        pltpu.VMEM_SHARED((NPAD, CH), jnp.float32),
        pltpu.SemaphoreType.DMA,
        pltpu.SemaphoreType.DMA,
        pltpu.SemaphoreType.DMA,
        pltpu.SemaphoreType.DMA,
    ],
)(_scat_body)


def _dinv_block(deg_ref):
    deg = deg_ref[0, :] + deg_ref[1, :] + 1.0
    return lax.rsqrt(deg)


def _prep_body(deg_ref, x_ref, w_ref, g_ref):
    dinv = _dinv_block(deg_ref)
    h = jnp.dot(x_ref[...], w_ref[...], preferred_element_type=jnp.float32,
                precision=lax.Precision.HIGHEST)
    g = h * dinv[:, None]
    g_ref[0] = g
    g_ref[1] = g


def _mid_body(parts_ref, deg_ref, b1_ref, w_ref, g2_ref):
    dinv = _dinv_block(deg_ref)
    s = parts_ref[0] + parts_ref[1]
    h = jnp.maximum(s * dinv[:, None] + b1_ref[...], 0.0)
    h2 = jnp.dot(h, w_ref[...], preferred_element_type=jnp.float32,
                 precision=lax.Precision.HIGHEST)
    g2 = h2 * dinv[:, None]
    g2_ref[0] = g2
    g2_ref[1] = g2


def _fin_body(parts_ref, deg_ref, b2_ref, z_ref):
    dinv = _dinv_block(deg_ref)
    s = parts_ref[0] + parts_ref[1]
    z_ref[...] = s * dinv[:, None] + b2_ref[...]


_row_spec = pl.BlockSpec((ROWB, CH), lambda i: (i, 0))
_deg_spec = pl.BlockSpec((NC, ROWB), lambda i: (0, i))
_parts_spec = pl.BlockSpec((NC, ROWB, CH), lambda i: (0, i, 0))
_wmat_spec = pl.BlockSpec((CH, CH), lambda i: (0, 0))
_bias_spec = pl.BlockSpec((1, CH), lambda i: (0, 0))

_prep_call = pl.pallas_call(
    _prep_body,
    grid=(GRID,),
    in_specs=[_deg_spec, _row_spec, _wmat_spec],
    out_specs=_parts_spec,
    out_shape=jax.ShapeDtypeStruct((NC, NPAD, CH), jnp.float32),
)

_mid_call = pl.pallas_call(
    _mid_body,
    grid=(GRID,),
    in_specs=[_parts_spec, _deg_spec, _bias_spec, _wmat_spec],
    out_specs=_parts_spec,
    out_shape=jax.ShapeDtypeStruct((NC, NPAD, CH), jnp.float32),
)

_fin_call = pl.pallas_call(
    _fin_body,
    grid=(GRID,),
    in_specs=[_parts_spec, _deg_spec, _bias_spec],
    out_specs=_row_spec,
    out_shape=jax.ShapeDtypeStruct((NPAD, CH), jnp.float32),
)


def kernel(x, edge_index, W1, b1, W2, b2):
    src = edge_index[0].astype(jnp.int32)
    dst = edge_index[1].astype(jnp.int32)
    pad = jnp.full((EP - N_EDGES,), N_NODES, jnp.int32)
    srcp = jnp.concatenate(
        [src, pad, jnp.full((2 * CHUNK,), N_NODES, jnp.int32)])
    dflat = jnp.concatenate([dst, pad])
    d0 = dflat[:CORE0_EDGES].reshape(NS, NCH0, CHUNK)
    d1 = dflat[CORE0_EDGES:].reshape(NS, NCH1, CHUNK)
    d1 = jnp.pad(d1, ((0, 0), (0, NCH0 - NCH1), (0, 0)),
                 constant_values=N_NODES)
    dsti = jnp.concatenate([d0, d1], axis=0)

    xp = jnp.pad(x, ((0, NPAD - N_NODES), (0, 0)))
    b1r = b1.reshape(1, CH)
    b2r = b2.reshape(1, CH)

    zblk = jnp.zeros((CHUNK, CH), jnp.float32)

    deg_parts = _deg_kernel(dflat)
    g1 = _prep_call(deg_parts, xp, W1)
    p1 = _scat_kernel(g1, srcp, dsti, zblk)
    g2 = _mid_call(p1, deg_parts, b1r, W2)
    p2 = _scat_kernel(g2, srcp, dsti, zblk)
    z = _fin_call(p2, deg_parts, b2r)
    return z[:N_NODES]


# speedup vs baseline: 1.2451x; 1.2451x over previous
import functools

import jax
import jax.numpy as jnp
from jax import lax
from jax.experimental import pallas as pl
from jax.experimental.pallas import tpu as pltpu
from jax.experimental.pallas import tpu_sc as plsc

N_NODES = 10000
CH = 128
N_EDGES = 320000

NC = 2
NS = 16
NW = NC * NS
CHUNK = 128
NCH = 79
TPW = NCH * CHUNK
EP = NW * TPW
NPAD = 10240
STRIPE = NPAD // NS
ROWB = 1024
GRID = NPAD // ROWB


def _sc_mesh():
    return plsc.VectorSubcoreMesh(core_axis_name="c", subcore_axis_name="s")


def _deg_body(dst_hbm, out_hbm, idx_v, ones_v, zer_v, acc, sem):
    cid = lax.axis_index("c")
    sid = lax.axis_index("s")
    wid = cid * NS + sid

    for i in range(CHUNK // 16):
        ones_v[pl.ds(i * 16, 16)] = jnp.ones((16,), jnp.float32)
    for i in range(STRIPE // 16):
        zer_v[pl.ds(i * 16, 16)] = jnp.zeros((16,), jnp.float32)

    pltpu.sync_copy(zer_v, acc.at[pl.ds(sid * STRIPE, STRIPE)])
    plsc.subcore_barrier()

    base = wid * TPW

    def chunk_step(j, carry):
        pltpu.sync_copy(dst_hbm.at[pl.ds(base + j * CHUNK, CHUNK)], idx_v)
        pltpu.sync_copy(ones_v, acc.at[idx_v], add=True)
        return carry

    lax.fori_loop(0, NCH, chunk_step, 0)
    plsc.subcore_barrier()
    pltpu.sync_copy(acc.at[pl.ds(sid * STRIPE, STRIPE)],
                    out_hbm.at[cid, pl.ds(sid * STRIPE, STRIPE)])


_deg_kernel = functools.partial(
    pl.kernel,
    out_type=jax.ShapeDtypeStruct((NC, NPAD), jnp.float32),
    mesh=_sc_mesh(),
    scratch_types=[
        pltpu.VMEM((CHUNK,), jnp.int32),
        pltpu.VMEM((CHUNK,), jnp.float32),
        pltpu.VMEM((STRIPE,), jnp.float32),
        pltpu.VMEM_SHARED((NPAD,), jnp.float32),
        pltpu.SemaphoreType.DMA,
    ],
)(_deg_body)


def _scat_body(g_hbm, src_hbm, dst_hbm, out_hbm, idxs_v, idxd_v, rows_v,
               acc, sem):
    cid = lax.axis_index("c")
    sid = lax.axis_index("s")
    wid = cid * NS + sid

    pltpu.sync_copy(g_hbm.at[pl.ds(sid * STRIPE, STRIPE)],
                    acc.at[pl.ds(sid * STRIPE, STRIPE)])
    plsc.subcore_barrier()

    base = wid * TPW

    def chunk_step(j, carry):
        pltpu.sync_copy(src_hbm.at[pl.ds(base + j * CHUNK, CHUNK)], idxs_v)
        pltpu.sync_copy(dst_hbm.at[pl.ds(base + j * CHUNK, CHUNK)], idxd_v)
        pltpu.async_copy(g_hbm.at[idxs_v], rows_v, sem).wait()
        pltpu.sync_copy(rows_v, acc.at[idxd_v], add=True)
        return carry

    lax.fori_loop(0, NCH, chunk_step, 0)
    plsc.subcore_barrier()
    pltpu.sync_copy(acc.at[pl.ds(sid * STRIPE, STRIPE)],
                    out_hbm.at[cid, pl.ds(sid * STRIPE, STRIPE)])


_scat_kernel = functools.partial(
    pl.kernel,
    out_type=jax.ShapeDtypeStruct((NC, NPAD, CH), jnp.float32),
    mesh=_sc_mesh(),
    scratch_types=[
        pltpu.VMEM((CHUNK,), jnp.int32),
        pltpu.VMEM((CHUNK,), jnp.int32),
        pltpu.VMEM((CHUNK, CH), jnp.float32),
        pltpu.VMEM_SHARED((NPAD, CH), jnp.float32),
        pltpu.SemaphoreType.DMA,
    ],
)(_scat_body)


def _dinv_block(deg_ref):
    deg = deg_ref[0, :] + deg_ref[1, :] + 1.0
    return lax.rsqrt(deg)


def _prep_body(deg_ref, x_ref, w_ref, g_ref):
    dinv = _dinv_block(deg_ref)
    h = jnp.dot(x_ref[...], w_ref[...], preferred_element_type=jnp.float32,
                precision=lax.Precision.HIGHEST)
    g_ref[...] = h * dinv[:, None]


def _mid_body(parts_ref, g1_ref, deg_ref, b1_ref, w_ref, g2_ref):
    dinv = _dinv_block(deg_ref)
    s = parts_ref[0] + parts_ref[1] - g1_ref[...]
    h = jnp.maximum(s * dinv[:, None] + b1_ref[...], 0.0)
    h2 = jnp.dot(h, w_ref[...], preferred_element_type=jnp.float32,
                 precision=lax.Precision.HIGHEST)
    g2_ref[...] = h2 * dinv[:, None]


def _fin_body(parts_ref, g2_ref, deg_ref, b2_ref, z_ref):
    dinv = _dinv_block(deg_ref)
    s = parts_ref[0] + parts_ref[1] - g2_ref[...]
    z_ref[...] = s * dinv[:, None] + b2_ref[...]


_row_spec = pl.BlockSpec((ROWB, CH), lambda i: (i, 0))
_deg_spec = pl.BlockSpec((NC, ROWB), lambda i: (0, i))
_parts_spec = pl.BlockSpec((NC, ROWB, CH), lambda i: (0, i, 0))
_wmat_spec = pl.BlockSpec((CH, CH), lambda i: (0, 0))
_bias_spec = pl.BlockSpec((1, CH), lambda i: (0, 0))

_prep_call = pl.pallas_call(
    _prep_body,
    grid=(GRID,),
    in_specs=[_deg_spec, _row_spec, _wmat_spec],
    out_specs=_row_spec,
    out_shape=jax.ShapeDtypeStruct((NPAD, CH), jnp.float32),
)

_mid_call = pl.pallas_call(
    _mid_body,
    grid=(GRID,),
    in_specs=[_parts_spec, _row_spec, _deg_spec, _bias_spec, _wmat_spec],
    out_specs=_row_spec,
    out_shape=jax.ShapeDtypeStruct((NPAD, CH), jnp.float32),
)

_fin_call = pl.pallas_call(
    _fin_body,
    grid=(GRID,),
    in_specs=[_parts_spec, _row_spec, _deg_spec, _bias_spec],
    out_specs=_row_spec,
    out_shape=jax.ShapeDtypeStruct((NPAD, CH), jnp.float32),
)


def kernel(x, edge_index, W1, b1, W2, b2):
    src = edge_index[0].astype(jnp.int32)
    dst = edge_index[1].astype(jnp.int32)
    pad = jnp.full((EP - N_EDGES,), N_NODES, jnp.int32)
    srcp = jnp.concatenate([src, pad])
    dstp = jnp.concatenate([dst, pad])

    xp = jnp.pad(x, ((0, NPAD - N_NODES), (0, 0)))
    b1r = b1.reshape(1, CH)
    b2r = b2.reshape(1, CH)

    deg_parts = _deg_kernel(dstp)
    g1 = _prep_call(deg_parts, xp, W1)
    p1 = _scat_kernel(g1, srcp, dstp)
    g2 = _mid_call(p1, g1, deg_parts, b1r, W2)
    p2 = _scat_kernel(g2, srcp, dstp)
    z = _fin_call(p2, g2, deg_parts, b2r)
    return z[:N_NODES]
